# baseline (speedup 1.0000x reference)
import functools
import jax
import jax.numpy as jnp
from jax import lax
from jax.experimental import pallas as pl
from jax.experimental.pallas import tpu as pltpu
from jax.experimental.pallas import tpu_sc as plsc

N = 4096
B1 = 256
B2 = 256
INT_BIG = 2**31 - 1

_SC_INFO = plsc.get_sparse_core_info()
NC = _SC_INFO.num_cores
NS = _SC_INFO.num_subcores
NW = NC * NS
WROWS = N // NW
CH = 8
NCHUNK = WROWS // CH
LANES = 16
VECS_PER_ROW = N // LANES
UNROLL = 4


@functools.partial(
    pl.kernel,
    out_type=[
        jax.ShapeDtypeStruct((N, LANES), jnp.float32),
        jax.ShapeDtypeStruct((NW, LANES), jnp.float32),
    ],
    mesh=plsc.VectorSubcoreMesh(core_axis_name="c", subcore_axis_name="s"),
    scratch_types=[
        pltpu.VMEM((CH, N), jnp.float32),
        pltpu.VMEM((CH, N), jnp.float32),
        pltpu.VMEM((CH, LANES), jnp.float32),
        pltpu.VMEM((LANES,), jnp.float32),
    ],
)
def _sc_pass1(adj_hbm, mg_hbm, degp_hbm, pminp_hbm, adj_v, mg_v, degp_v, min_v):
    wid = lax.axis_index("s") * NC + lax.axis_index("c")
    base = wid * WROWS

    def chunk_body(ch, minv):
        row0 = base + ch * CH
        pltpu.sync_copy(adj_hbm.at[pl.ds(row0, CH)], adj_v)
        pltpu.sync_copy(mg_hbm.at[pl.ds(row0, CH)], mg_v)
        for r in range(CH):
            def inner(k, carry):
                rs, mv = carry
                for u in range(UNROLL):
                    off = (k * UNROLL + u) * LANES
                    a = adj_v[r, pl.ds(off, LANES)]
                    m = mg_v[r, pl.ds(off, LANES)]
                    mv = jnp.minimum(mv, m * (1.0 - 2.0 * a))
                    rs = rs + a
                return rs, mv

            rs, minv = lax.fori_loop(0, VECS_PER_ROW // UNROLL, inner,
                                     (jnp.zeros((LANES,), jnp.float32), minv))
            degp_v[r] = rs
        pltpu.sync_copy(degp_v, degp_hbm.at[pl.ds(row0, CH)])
        return minv

    minv = lax.fori_loop(0, NCHUNK, chunk_body,
                         jnp.full((LANES,), jnp.inf, jnp.float32))
    min_v[...] = minv
    pltpu.sync_copy(min_v, pminp_hbm.at[wid])


def _p2a_body(adj_ref, acr_ref, acc_ref, am_ref, adjnew_ref):
    i = pl.program_id(0)
    a = adj_ref[...]
    acr = acr_ref[...]
    acc = acc_ref[...]
    adjnew_ref[...] = a

    rows = lax.broadcasted_iota(jnp.int32, (B2, N), 0) + i * B2
    cols = lax.broadcasted_iota(jnp.int32, (B2, N), 1)

    acs = acr + jnp.transpose(acc)
    acs = jnp.where(rows == cols, 0.0, acs)
    acs = jnp.clip(acs, -1.0, 1.0)
    am_ref[...] = jnp.clip(a + acs, 0.0, 1.0)


def _p2b_body(adj_ref, mg_ref, degp_ref, pminp_ref, ms_ref, bestv_ref, besti_ref):
    i = pl.program_id(0)
    a = adj_ref[...]
    mg = mg_ref[...]

    rows = lax.broadcasted_iota(jnp.int32, (B1, N), 0) + i * B1
    cols = lax.broadcasted_iota(jnp.int32, (B1, N), 1)

    pmin = jnp.min(pminp_ref[...])
    deg = jnp.sum(degp_ref[...], axis=1)
    d1c = (deg == 1.0).astype(jnp.float32)
    d1r = (jnp.sum(degp_ref[pl.ds(i * B1, B1), :], axis=1)
           == 1.0).astype(jnp.float32)
    maskv = a * (d1r[:, None] + d1c[None, :])
    s2 = mg * (1.0 - 2.0 * a) - pmin
    ms = s2 * maskv
    ms_ref[...] = ms

    tmax = jnp.max(ms)
    cand = jnp.min(jnp.where(ms == tmax, rows * N + cols, INT_BIG))

    @pl.when(i == 0)
    def _():
        bestv_ref[0, 0] = -1.0
        besti_ref[0, 0] = 0

    @pl.when(tmax > bestv_ref[0, 0])
    def _():
        bestv_ref[0, 0] = tmax
        besti_ref[0, 0] = cand


def _flip_body(pos_ref, nv_ref, adjin_ref, out_ref):
    k = pl.program_id(0)
    r0 = (pos_ref[k, 0] // 8) * 8
    c0 = (pos_ref[k, 1] // 128) * 128
    r = pos_ref[0, 0]
    c = pos_ref[0, 1]
    rows = lax.broadcasted_iota(jnp.int32, (8, 128), 0) + r0
    cols = lax.broadcasted_iota(jnp.int32, (8, 128), 1) + c0
    hit = ((rows == r) & (cols == c)) | ((rows == c) & (cols == r))
    out_ref[...] = jnp.where(hit, nv_ref[0, 0], adjin_ref[...])


def kernel(adj, adj_changes, meta_grad, feature_matrix, labels, train_ids, val_ids):
    del feature_matrix, labels, train_ids, val_ids

    degp, pminp = _sc_pass1(adj, meta_grad)

    adj_modified, adj_new0 = pl.pallas_call(
        _p2a_body,
        grid=(N // B2,),
        in_specs=[
            pl.BlockSpec((B2, N), lambda i: (i, 0)),
            pl.BlockSpec((B2, N), lambda i: (i, 0)),
            pl.BlockSpec((N, B2), lambda i: (0, i)),
        ],
        out_specs=[
            pl.BlockSpec((B2, N), lambda i: (i, 0)),
            pl.BlockSpec((B2, N), lambda i: (i, 0)),
        ],
        out_shape=[
            jax.ShapeDtypeStruct((N, N), jnp.float32),
            jax.ShapeDtypeStruct((N, N), jnp.float32),
        ],
    )(adj, adj_changes, adj_changes)

    masked_scores, bestv, besti = pl.pallas_call(
        _p2b_body,
        grid=(N // B1,),
        in_specs=[
            pl.BlockSpec((B1, N), lambda i: (i, 0)),
            pl.BlockSpec((B1, N), lambda i: (i, 0)),
            pl.BlockSpec((N, LANES), lambda i: (0, 0)),
            pl.BlockSpec((NW, LANES), lambda i: (0, 0)),
        ],
        out_specs=[
            pl.BlockSpec((B1, N), lambda i: (i, 0)),
            pl.BlockSpec(memory_space=pltpu.SMEM),
            pl.BlockSpec(memory_space=pltpu.SMEM),
        ],
        out_shape=[
            jax.ShapeDtypeStruct((N, N), jnp.float32),
            jax.ShapeDtypeStruct((1, 1), jnp.float32),
            jax.ShapeDtypeStruct((1, 1), jnp.int32),
        ],
    )(adj, meta_grad, degp, pminp)

    flat = besti[0, 0]
    r = flat // N
    c = flat % N
    pos = jnp.stack([jnp.stack([r, c]), jnp.stack([c, r])]).astype(jnp.int32)
    new_val = jnp.where(bestv[0, 0] > 0.0, 0.0, 1.0).reshape(1, 1).astype(jnp.float32)

    adj_new = pl.pallas_call(
        _flip_body,
        grid_spec=pltpu.PrefetchScalarGridSpec(
            num_scalar_prefetch=1,
            grid=(2,),
            in_specs=[
                pl.BlockSpec(memory_space=pltpu.SMEM),
                pl.BlockSpec((8, 128), lambda k, pos_ref: (pos_ref[k, 0] // 8, pos_ref[k, 1] // 128)),
            ],
            out_specs=pl.BlockSpec((8, 128), lambda k, pos_ref: (pos_ref[k, 0] // 8, pos_ref[k, 1] // 128)),
        ),
        out_shape=jax.ShapeDtypeStruct((N, N), jnp.float32),
        input_output_aliases={2: 0},
    )(pos, new_val, adj_new0)

    return adj_new, adj_modified, masked_scores


# speedup vs baseline: 1.0264x; 1.0264x over previous
import functools
import jax
import jax.numpy as jnp
from jax import lax
from jax.experimental import pallas as pl
from jax.experimental.pallas import tpu as pltpu
from jax.experimental.pallas import tpu_sc as plsc

N = 4096
B1 = 256
B2 = 256
INT_BIG = 2**31 - 1

_SC_INFO = plsc.get_sparse_core_info()
NC = _SC_INFO.num_cores
NS = _SC_INFO.num_subcores
NW = NC * NS
WROWS = N // NW
CH = 4
NCHUNK = WROWS // CH
LANES = 16
VECS_PER_ROW = N // LANES
UNROLL = 8


@functools.partial(
    pl.kernel,
    out_type=[
        jax.ShapeDtypeStruct((N, LANES), jnp.float32),
        jax.ShapeDtypeStruct((NW, LANES), jnp.float32),
    ],
    mesh=plsc.VectorSubcoreMesh(core_axis_name="c", subcore_axis_name="s"),
    scratch_types=[
        pltpu.VMEM((CH, N), jnp.float32),
        pltpu.VMEM((CH, N), jnp.float32),
        pltpu.VMEM((CH, N), jnp.float32),
        pltpu.VMEM((CH, N), jnp.float32),
        pltpu.VMEM((CH, LANES), jnp.float32),
        pltpu.VMEM((LANES,), jnp.float32),
        pltpu.SemaphoreType.DMA,
        pltpu.SemaphoreType.DMA,
    ],
)
def _sc_pass1(adj_hbm, mg_hbm, degp_hbm, pminp_hbm,
              a0_v, m0_v, a1_v, m1_v, degp_v, min_v, s0, s1):
    wid = lax.axis_index("s") * NC + lax.axis_index("c")
    base = wid * WROWS
    abufs = (a0_v, a1_v)
    mbufs = (m0_v, m1_v)
    sems = (s0, s1)

    def start_in(ch, slot):
        row0 = base + ch * CH
        pltpu.async_copy(adj_hbm.at[pl.ds(row0, CH)], abufs[slot], sems[slot])
        pltpu.async_copy(mg_hbm.at[pl.ds(row0, CH)], mbufs[slot], sems[slot])

    def wait_in(slot):
        pltpu.make_async_copy(adj_hbm.at[pl.ds(0, CH)], abufs[slot], sems[slot]).wait()
        pltpu.make_async_copy(mg_hbm.at[pl.ds(0, CH)], mbufs[slot], sems[slot]).wait()

    start_in(0, 0)

    def outer(g, minv):
        for b in range(2):
            ch = 2 * g + b
            wait_in(b)

            @pl.when(ch + 1 < NCHUNK)
            def _():
                start_in(ch + 1, 1 - b)

            for r in range(CH):
                def inner(k, carry):
                    rs, mv = carry
                    for u in range(UNROLL):
                        off = (k * UNROLL + u) * LANES
                        a = abufs[b][r, pl.ds(off, LANES)]
                        m = mbufs[b][r, pl.ds(off, LANES)]
                        mv = jnp.minimum(mv, m * (1.0 - 2.0 * a))
                        rs = rs + a
                    return rs, mv

                rs, minv = lax.fori_loop(0, VECS_PER_ROW // UNROLL, inner,
                                         (jnp.zeros((LANES,), jnp.float32), minv))
                degp_v[r] = rs
            pltpu.sync_copy(degp_v, degp_hbm.at[pl.ds(base + ch * CH, CH)])
        return minv

    minv = lax.fori_loop(0, NCHUNK // 2, outer,
                         jnp.full((LANES,), jnp.inf, jnp.float32))
    min_v[...] = minv
    pltpu.sync_copy(min_v, pminp_hbm.at[wid])


def _p2a_body(adj_ref, acr_ref, acc_ref, am_ref, adjnew_ref):
    i = pl.program_id(0)
    a = adj_ref[...]
    acr = acr_ref[...]
    acc = acc_ref[...]
    adjnew_ref[...] = a

    rows = lax.broadcasted_iota(jnp.int32, (B2, N), 0) + i * B2
    cols = lax.broadcasted_iota(jnp.int32, (B2, N), 1)

    acs = acr + jnp.transpose(acc)
    acs = jnp.where(rows == cols, 0.0, acs)
    acs = jnp.clip(acs, -1.0, 1.0)
    am_ref[...] = jnp.clip(a + acs, 0.0, 1.0)


def _p2b_body(adj_ref, mg_ref, degp_ref, pminp_ref, ms_ref, bestv_ref, besti_ref):
    i = pl.program_id(0)
    a = adj_ref[...]
    mg = mg_ref[...]

    rows = lax.broadcasted_iota(jnp.int32, (B1, N), 0) + i * B1
    cols = lax.broadcasted_iota(jnp.int32, (B1, N), 1)

    pmin = jnp.min(pminp_ref[...])
    deg = jnp.sum(degp_ref[...], axis=1)
    d1c = (deg == 1.0).astype(jnp.float32)
    d1r = (jnp.sum(degp_ref[pl.ds(i * B1, B1), :], axis=1)
           == 1.0).astype(jnp.float32)
    maskv = a * (d1r[:, None] + d1c[None, :])
    s2 = mg * (1.0 - 2.0 * a) - pmin
    ms = s2 * maskv
    ms_ref[...] = ms

    tmax = jnp.max(ms)
    cand = jnp.min(jnp.where(ms == tmax, rows * N + cols, INT_BIG))

    @pl.when(i == 0)
    def _():
        bestv_ref[0, 0] = -1.0
        besti_ref[0, 0] = 0

    @pl.when(tmax > bestv_ref[0, 0])
    def _():
        bestv_ref[0, 0] = tmax
        besti_ref[0, 0] = cand


def _flip_body(pos_ref, nv_ref, adjin_ref, out_ref):
    k = pl.program_id(0)
    r0 = (pos_ref[k, 0] // 8) * 8
    c0 = (pos_ref[k, 1] // 128) * 128
    r = pos_ref[0, 0]
    c = pos_ref[0, 1]
    rows = lax.broadcasted_iota(jnp.int32, (8, 128), 0) + r0
    cols = lax.broadcasted_iota(jnp.int32, (8, 128), 1) + c0
    hit = ((rows == r) & (cols == c)) | ((rows == c) & (cols == r))
    out_ref[...] = jnp.where(hit, nv_ref[0, 0], adjin_ref[...])


def kernel(adj, adj_changes, meta_grad, feature_matrix, labels, train_ids, val_ids):
    del feature_matrix, labels, train_ids, val_ids

    degp, pminp = _sc_pass1(adj, meta_grad)

    adj_modified, adj_new0 = pl.pallas_call(
        _p2a_body,
        grid=(N // B2,),
        in_specs=[
            pl.BlockSpec((B2, N), lambda i: (i, 0)),
            pl.BlockSpec((B2, N), lambda i: (i, 0)),
            pl.BlockSpec((N, B2), lambda i: (0, i)),
        ],
        out_specs=[
            pl.BlockSpec((B2, N), lambda i: (i, 0)),
            pl.BlockSpec((B2, N), lambda i: (i, 0)),
        ],
        out_shape=[
            jax.ShapeDtypeStruct((N, N), jnp.float32),
            jax.ShapeDtypeStruct((N, N), jnp.float32),
        ],
    )(adj, adj_changes, adj_changes)

    masked_scores, bestv, besti = pl.pallas_call(
        _p2b_body,
        grid=(N // B1,),
        in_specs=[
            pl.BlockSpec((B1, N), lambda i: (i, 0)),
            pl.BlockSpec((B1, N), lambda i: (i, 0)),
            pl.BlockSpec((N, LANES), lambda i: (0, 0)),
            pl.BlockSpec((NW, LANES), lambda i: (0, 0)),
        ],
        out_specs=[
            pl.BlockSpec((B1, N), lambda i: (i, 0)),
            pl.BlockSpec(memory_space=pltpu.SMEM),
            pl.BlockSpec(memory_space=pltpu.SMEM),
        ],
        out_shape=[
            jax.ShapeDtypeStruct((N, N), jnp.float32),
            jax.ShapeDtypeStruct((1, 1), jnp.float32),
            jax.ShapeDtypeStruct((1, 1), jnp.int32),
        ],
    )(adj, meta_grad, degp, pminp)

    flat = besti[0, 0]
    r = flat // N
    c = flat % N
    pos = jnp.stack([jnp.stack([r, c]), jnp.stack([c, r])]).astype(jnp.int32)
    new_val = jnp.where(bestv[0, 0] > 0.0, 0.0, 1.0).reshape(1, 1).astype(jnp.float32)

    adj_new = pl.pallas_call(
        _flip_body,
        grid_spec=pltpu.PrefetchScalarGridSpec(
            num_scalar_prefetch=1,
            grid=(2,),
            in_specs=[
                pl.BlockSpec(memory_space=pltpu.SMEM),
                pl.BlockSpec((8, 128), lambda k, pos_ref: (pos_ref[k, 0] // 8, pos_ref[k, 1] // 128)),
            ],
            out_specs=pl.BlockSpec((8, 128), lambda k, pos_ref: (pos_ref[k, 0] // 8, pos_ref[k, 1] // 128)),
        ),
        out_shape=jax.ShapeDtypeStruct((N, N), jnp.float32),
        input_output_aliases={2: 0},
    )(pos, new_val, adj_new0)

    return adj_new, adj_modified, masked_scores


# speedup vs baseline: 1.2911x; 1.2579x over previous
import jax
import jax.numpy as jnp
from jax import lax
from jax.experimental import pallas as pl
from jax.experimental.pallas import tpu as pltpu

N = 4096
B1 = 512
B2 = 256
INT_BIG = 2**31 - 1


def _pass1_body(adj_ref, mg_ref, adjnew_ref, deg_ref, pmin_ref):
    i = pl.program_id(0)
    a = adj_ref[...]
    m = mg_ref[...]
    adjnew_ref[...] = a
    deg_ref[0, pl.ds(i * B1, B1)] = jnp.sum(a, axis=1)
    tmin = jnp.min(m * (1.0 - 2.0 * a))

    @pl.when(i == 0)
    def _():
        pmin_ref[0, 0] = tmin

    @pl.when(i > 0)
    def _():
        pmin_ref[0, 0] = jnp.minimum(pmin_ref[0, 0], tmin)


def _pass2_body(adj_ref, acr_ref, acc_ref, mg_ref, deg_ref, pmin_ref,
                am_ref, ms_ref, bestv_ref, besti_ref):
    i = pl.program_id(0)
    a = adj_ref[...]
    acr = acr_ref[...]
    acc = acc_ref[...]
    mg = mg_ref[...]

    rows = lax.broadcasted_iota(jnp.int32, (B2, N), 0) + i * B2
    cols = lax.broadcasted_iota(jnp.int32, (B2, N), 1)

    acs = acr + jnp.transpose(acc)
    acs = jnp.where(rows == cols, 0.0, acs)
    acs = jnp.clip(acs, -1.0, 1.0)
    am_ref[...] = jnp.clip(a + acs, 0.0, 1.0)

    deg = deg_ref[0, :]
    d1c = (deg == 1.0).astype(jnp.float32)
    d1r = (deg_ref[0, pl.ds(i * B2, B2)] == 1.0).astype(jnp.float32)
    maskv = a * (d1r[:, None] + d1c[None, :])
    s2 = mg * (1.0 - 2.0 * a) - pmin_ref[0, 0]
    ms = s2 * maskv
    ms_ref[...] = ms

    tmax = jnp.max(ms)
    cand = jnp.min(jnp.where(ms == tmax, rows * N + cols, INT_BIG))

    @pl.when(i == 0)
    def _():
        bestv_ref[0, 0] = -1.0
        besti_ref[0, 0] = 0

    @pl.when(tmax > bestv_ref[0, 0])
    def _():
        bestv_ref[0, 0] = tmax
        besti_ref[0, 0] = cand


def _flip_body(pos_ref, nv_ref, adjin_ref, out_ref):
    k = pl.program_id(0)
    r0 = (pos_ref[k, 0] // 8) * 8
    c0 = (pos_ref[k, 1] // 128) * 128
    r = pos_ref[0, 0]
    c = pos_ref[0, 1]
    rows = lax.broadcasted_iota(jnp.int32, (8, 128), 0) + r0
    cols = lax.broadcasted_iota(jnp.int32, (8, 128), 1) + c0
    hit = ((rows == r) & (cols == c)) | ((rows == c) & (cols == r))
    out_ref[...] = jnp.where(hit, nv_ref[0, 0], adjin_ref[...])


def kernel(adj, adj_changes, meta_grad, feature_matrix, labels, train_ids, val_ids):
    del feature_matrix, labels, train_ids, val_ids

    adj_new0, deg, pmin = pl.pallas_call(
        _pass1_body,
        grid=(N // B1,),
        in_specs=[
            pl.BlockSpec((B1, N), lambda i: (i, 0)),
            pl.BlockSpec((B1, N), lambda i: (i, 0)),
        ],
        out_specs=[
            pl.BlockSpec((B1, N), lambda i: (i, 0)),
            pl.BlockSpec((1, N), lambda i: (0, 0)),
            pl.BlockSpec(memory_space=pltpu.SMEM),
        ],
        out_shape=[
            jax.ShapeDtypeStruct((N, N), jnp.float32),
            jax.ShapeDtypeStruct((1, N), jnp.float32),
            jax.ShapeDtypeStruct((1, 1), jnp.float32),
        ],
    )(adj, meta_grad)

    adj_modified, masked_scores, bestv, besti = pl.pallas_call(
        _pass2_body,
        grid=(N // B2,),
        in_specs=[
            pl.BlockSpec((B2, N), lambda i: (i, 0)),
            pl.BlockSpec((B2, N), lambda i: (i, 0)),
            pl.BlockSpec((N, B2), lambda i: (0, i)),
            pl.BlockSpec((B2, N), lambda i: (i, 0)),
            pl.BlockSpec((1, N), lambda i: (0, 0)),
            pl.BlockSpec(memory_space=pltpu.SMEM),
        ],
        out_specs=[
            pl.BlockSpec((B2, N), lambda i: (i, 0)),
            pl.BlockSpec((B2, N), lambda i: (i, 0)),
            pl.BlockSpec(memory_space=pltpu.SMEM),
            pl.BlockSpec(memory_space=pltpu.SMEM),
        ],
        out_shape=[
            jax.ShapeDtypeStruct((N, N), jnp.float32),
            jax.ShapeDtypeStruct((N, N), jnp.float32),
            jax.ShapeDtypeStruct((1, 1), jnp.float32),
            jax.ShapeDtypeStruct((1, 1), jnp.int32),
        ],
    )(adj, adj_changes, adj_changes, meta_grad, deg, pmin)

    flat = besti[0, 0]
    r = flat // N
    c = flat % N
    pos = jnp.stack([jnp.stack([r, c]), jnp.stack([c, r])]).astype(jnp.int32)
    new_val = jnp.where(bestv[0, 0] > 0.0, 0.0, 1.0).reshape(1, 1).astype(jnp.float32)

    adj_new = pl.pallas_call(
        _flip_body,
        grid_spec=pltpu.PrefetchScalarGridSpec(
            num_scalar_prefetch=1,
            grid=(2,),
            in_specs=[
                pl.BlockSpec(memory_space=pltpu.SMEM),
                pl.BlockSpec((8, 128), lambda k, pos_ref: (pos_ref[k, 0] // 8, pos_ref[k, 1] // 128)),
            ],
            out_specs=pl.BlockSpec((8, 128), lambda k, pos_ref: (pos_ref[k, 0] // 8, pos_ref[k, 1] // 128)),
        ),
        out_shape=jax.ShapeDtypeStruct((N, N), jnp.float32),
        input_output_aliases={2: 0},
    )(pos, new_val, adj_new0)

    return adj_new, adj_modified, masked_scores
